# Initial kernel scaffold; baseline (speedup 1.0000x reference)
import jax
import jax.numpy as jnp
from jax.experimental import pallas as pl


def kernel(x, edge_index, edge_attr, batch, atom_origin_type, W_ei, b_ei, W_conv, b_conv, W_en, b_en, W_f1, b_f1, W_f2, b_f2):
    raise NotImplementedError("write your pallas kernel here")


# speedup vs baseline: 2.3837x; 2.3837x over previous
import functools
import jax
import jax.numpy as jnp
from jax import lax
from jax.experimental import pallas as pl
from jax.experimental.pallas import tpu as pltpu
from jax.experimental.pallas import tpu_sc as plsc

N = 10000
E = 320000
HID = 128
NG = 64
CH = 128
NCH = E // CH
NP = 10112
NTB = NP // CH
NSC = 2
NSUB = 16
R = 2000
GRID_E = E // R


def _worker_range(wid, total, nworkers):
  base = total // nworkers
  rem = total % nworkers
  count = base + jnp.where(wid < rem, 1, 0)
  start = wid * base + jnp.minimum(wid, rem)
  return start, count


def _sc_scatter_body(z_hbm, col_hbm, zeros_hbm, part_hbm, tbl, zbuf, idx_v):
  c = lax.axis_index("c")
  s = lax.axis_index("s")
  pltpu.sync_copy(zeros_hbm, zbuf)
  for k in range(5):
    ci = s + NSUB * k
    @pl.when(ci < NTB)
    def _():
      pltpu.sync_copy(zbuf, tbl.at[pl.ds(ci * CH, CH)])
  plsc.subcore_barrier()
  start, count = _worker_range(s, NCH // NSC, NSUB)
  start = start + c * (NCH // NSC)

  def chunk(k, _):
    ci = start + k
    pltpu.sync_copy(col_hbm.at[ci], idx_v)
    pltpu.sync_copy(z_hbm.at[pl.ds(ci * CH, CH)], zbuf)
    pltpu.sync_copy(zbuf, tbl.at[idx_v], add=True)
    return 0

  lax.fori_loop(0, count, chunk, 0)
  plsc.subcore_barrier()
  for k in range(5):
    ci = s + NSUB * k
    @pl.when(ci < NTB)
    def _():
      pltpu.sync_copy(tbl.at[pl.ds(ci * CH, CH)], part_hbm.at[c, pl.ds(ci * CH, CH)])


_sc_scatter = pl.kernel(
    _sc_scatter_body,
    out_type=jax.ShapeDtypeStruct((NSC, NP, HID), jnp.float32),
    mesh=plsc.VectorSubcoreMesh(core_axis_name="c", subcore_axis_name="s", num_cores=NSC, num_subcores=NSUB),
    scratch_types=[
        pltpu.VMEM_SHARED((NP, HID), jnp.float32),
        pltpu.VMEM((CH, HID), jnp.float32),
        pltpu.VMEM((CH,), jnp.int32),
    ],
)


def _sc_gather_body(tbl_hbm, idx_hbm, out_hbm, idx_v, rows_v, sem):
  c = lax.axis_index("c")
  s = lax.axis_index("s")
  wid = s * NSC + c
  start, count = _worker_range(wid, NCH, NSC * NSUB)

  def chunk(k, _):
    ci = start + k
    pltpu.sync_copy(idx_hbm.at[ci], idx_v)
    pltpu.async_copy(tbl_hbm.at[idx_v], rows_v, sem).wait()
    pltpu.sync_copy(rows_v, out_hbm.at[pl.ds(ci * CH, CH)])
    return 0

  lax.fori_loop(0, count, chunk, 0)


_sc_gather = pl.kernel(
    _sc_gather_body,
    out_type=jax.ShapeDtypeStruct((E, HID), jnp.float32),
    mesh=plsc.VectorSubcoreMesh(core_axis_name="c", subcore_axis_name="s", num_cores=NSC, num_subcores=NSUB),
    scratch_types=[
        pltpu.VMEM((CH,), jnp.int32),
        pltpu.VMEM((CH, HID), jnp.float32),
        pltpu.SemaphoreType.DMA,
    ],
)


def _pairswap(z):
  even = lax.broadcasted_iota(jnp.int32, (z.shape[0], 1), 0) % 2 == 0
  n = z.shape[0]
  return jnp.where(even, pltpu.roll(z, n - 1, 0), pltpu.roll(z, 1, 0))


def _xw_body(x_ref, w_ref, o_ref):
  o_ref[...] = jnp.dot(x_ref[...], w_ref[...],
                       preferred_element_type=jnp.float32)


def _k_xw(x, wx):
  return pl.pallas_call(
      _xw_body,
      grid=(N // R,),
      in_specs=[pl.BlockSpec((R, HID), lambda i: (i, 0)),
                pl.BlockSpec((HID, HID), lambda i: (0, 0))],
      out_specs=pl.BlockSpec((R, HID), lambda i: (i, 0)),
      out_shape=jax.ShapeDtypeStruct((N, HID), jnp.float32),
  )(x, wx)


def _k1_body(g0_ref, ea_ref, we_ref, bei_ref, w0_ref, z_ref, h0_ref):
  m = jnp.dot(ea_ref[...], we_ref[...], preferred_element_type=jnp.float32)
  h0 = jnp.maximum(g0_ref[...] + m + bei_ref[...], 0.0)
  h0_ref[...] = h0
  z_ref[...] = jnp.dot(h0, w0_ref[...], preferred_element_type=jnp.float32)


def _k1(g0, ea, we, bei, w0):
  return pl.pallas_call(
      _k1_body,
      grid=(GRID_E,),
      in_specs=[pl.BlockSpec((R, HID), lambda i: (i, 0)),
                pl.BlockSpec((R, 16), lambda i: (i, 0)),
                pl.BlockSpec((16, HID), lambda i: (0, 0)),
                pl.BlockSpec((1, HID), lambda i: (0, 0)),
                pl.BlockSpec((HID, HID), lambda i: (0, 0))],
      out_specs=[pl.BlockSpec((R, HID), lambda i: (i, 0)),
                 pl.BlockSpec((R, HID), lambda i: (i, 0))],
      out_shape=[jax.ShapeDtypeStruct((E, HID), jnp.float32),
                 jax.ShapeDtypeStruct((E, HID), jnp.float32)],
  )(g0, ea, we, bei, w0)


def _layer_body(g_ref, z_ref, h0_ref, b_ref, w_ref, zo_ref):
  h = jnp.maximum(g_ref[...] - _pairswap(z_ref[...]) + h0_ref[...]
                  + b_ref[...], 0.0)
  zo_ref[...] = jnp.dot(h, w_ref[...], preferred_element_type=jnp.float32)


def _k_layer(g, z, h0, b, w):
  return pl.pallas_call(
      _layer_body,
      grid=(GRID_E,),
      in_specs=[pl.BlockSpec((R, HID), lambda i: (i, 0)),
                pl.BlockSpec((R, HID), lambda i: (i, 0)),
                pl.BlockSpec((R, HID), lambda i: (i, 0)),
                pl.BlockSpec((1, HID), lambda i: (0, 0)),
                pl.BlockSpec((HID, HID), lambda i: (0, 0))],
      out_specs=pl.BlockSpec((R, HID), lambda i: (i, 0)),
      out_shape=jax.ShapeDtypeStruct((E, HID), jnp.float32),
  )(g, z, h0, b, w)


def _hlast_body(g_ref, z_ref, h0_ref, b_ref, h_ref):
  h_ref[...] = jnp.maximum(g_ref[...] - _pairswap(z_ref[...]) + h0_ref[...]
                           + b_ref[...], 0.0)


def _k_hlast(g, z, h0, b):
  return pl.pallas_call(
      _hlast_body,
      grid=(GRID_E,),
      in_specs=[pl.BlockSpec((R, HID), lambda i: (i, 0)),
                pl.BlockSpec((R, HID), lambda i: (i, 0)),
                pl.BlockSpec((R, HID), lambda i: (i, 0)),
                pl.BlockSpec((1, HID), lambda i: (0, 0))],
      out_specs=pl.BlockSpec((R, HID), lambda i: (i, 0)),
      out_shape=jax.ShapeDtypeStruct((E, HID), jnp.float32),
  )(g, z, h0, b)


def _add_body(a_ref, b_ref, o_ref):
  o_ref[...] = a_ref[0] + b_ref[0]


def _k_addparts(parts):
  return pl.pallas_call(
      _add_body,
      grid=(NTB,),
      in_specs=[pl.BlockSpec((1, CH, HID), lambda i: (0, i, 0)),
                pl.BlockSpec((1, CH, HID), lambda i: (1, i, 0))],
      out_specs=pl.BlockSpec((CH, HID), lambda i: (i, 0)),
      out_shape=jax.ShapeDtypeStruct((NP, HID), jnp.float32),
  )(parts, parts)


def _tail_body(x_ref, s_ref, batch_ref, wnx_ref, wns_ref, ben_ref,
               wf1_ref, bf1_ref, wf2_ref, bf2_ref, o_ref, acc):
  i = pl.program_id(0)

  @pl.when(i == 0)
  def _():
    acc[...] = jnp.zeros_like(acc)

  hn = jnp.maximum(
      jnp.dot(x_ref[...], wnx_ref[...], preferred_element_type=jnp.float32)
      + jnp.dot(s_ref[...], wns_ref[...], preferred_element_type=jnp.float32)
      + ben_ref[...], 0.0)
  bm = batch_ref[0]
  oh = (bm == lax.broadcasted_iota(jnp.int32, (NG, 1), 0)).astype(jnp.float32)
  acc[...] += jnp.dot(oh, hn, preferred_element_type=jnp.float32)

  @pl.when(i == pl.num_programs(0) - 1)
  def _():
    v = jnp.maximum(
        jnp.dot(acc[...], wf1_ref[...], preferred_element_type=jnp.float32)
        + bf1_ref[...], 0.0)
    o_ref[...] = (jnp.sum(v * wf2_ref[...], axis=1, keepdims=True)
                  + bf2_ref[...])


def _k_tail(x, s, batch3, wnx, wns, ben, wf1, bf1, wf2t, bf2):
  return pl.pallas_call(
      _tail_body,
      grid=(N // R,),
      in_specs=[pl.BlockSpec((R, HID), lambda i: (i, 0)),
                pl.BlockSpec((R, HID), lambda i: (i, 0)),
                pl.BlockSpec((1, 1, R), lambda i: (i, 0, 0)),
                pl.BlockSpec((HID, HID), lambda i: (0, 0)),
                pl.BlockSpec((HID, HID), lambda i: (0, 0)),
                pl.BlockSpec((1, HID), lambda i: (0, 0)),
                pl.BlockSpec((HID, HID), lambda i: (0, 0)),
                pl.BlockSpec((1, HID), lambda i: (0, 0)),
                pl.BlockSpec((1, HID), lambda i: (0, 0)),
                pl.BlockSpec((1, 1), lambda i: (0, 0))],
      out_specs=pl.BlockSpec((NG, 1), lambda i: (0, 0)),
      out_shape=jax.ShapeDtypeStruct((NG, 1), jnp.float32),
      scratch_shapes=[pltpu.VMEM((NG, HID), jnp.float32)],
  )(x, s, batch3, wnx, wns, ben, wf1, bf1, wf2t, bf2)


@jax.jit
def kernel(x, edge_index, edge_attr, batch, atom_origin_type,
           W_ei, b_ei, W_conv, b_conv, W_en, b_en, W_f1, b_f1, W_f2, b_f2):
  del atom_origin_type
  row2d = edge_index[0].reshape(NCH, CH)
  col2d = edge_index[1].reshape(NCH, CH)
  zeros = jnp.zeros((CH, HID), jnp.float32)
  wx = W_ei[:HID]
  we = W_ei[HID:]
  bei = b_ei.reshape(1, HID)
  batch3 = batch.reshape(N // R, 1, R)

  xw = _k_xw(x, wx)
  g0 = _sc_gather(xw, row2d)
  z, h0 = _k1(g0, edge_attr, we, bei, W_conv[0])
  for l in range(3):
    parts = _sc_scatter(z, col2d, zeros)
    t = _k_addparts(parts)
    g = _sc_gather(t, row2d)
    b_l = b_conv[l].reshape(1, HID)
    if l < 2:
      z = _k_layer(g, z, h0, b_l, W_conv[l + 1])
    else:
      h3 = _k_hlast(g, z, h0, b_l)
  parts = _sc_scatter(h3, col2d, zeros)
  s = _k_addparts(parts)
  out = _k_tail(x, s[:N], batch3, W_en[:HID], W_en[HID:], b_en.reshape(1, HID),
                W_f1, b_f1.reshape(1, HID), W_f2.reshape(1, HID),
                b_f2.reshape(1, 1))
  return out.reshape(NG)
